# baseline (speedup 1.0000x reference)
import functools

import jax
import jax.numpy as jnp
from jax import lax
from jax.experimental import pallas as pl
from jax.experimental.pallas import tpu as pltpu
from jax.experimental.pallas import tpu_sc as plsc

_NUM_CORES = 2
_NUM_SUBCORES = 16
_NUM_WORKERS = _NUM_CORES * _NUM_SUBCORES


@functools.partial(jax.jit, static_argnames=("n", "d"))
def _sc_gather(ids_flat, table, *, n, d):
    bpw = n // _NUM_WORKERS
    chunk = 16
    nchunk = bpw // chunk
    nbuf = 3
    mesh = plsc.VectorSubcoreMesh(core_axis_name="c", subcore_axis_name="s")

    @functools.partial(
        pl.kernel,
        mesh=mesh,
        out_type=jax.ShapeDtypeStruct((n, d), table.dtype),
        scratch_types=[
            pltpu.VMEM((bpw,), jnp.int32),
        ] + [pltpu.VMEM((chunk, d), table.dtype)] * nbuf
          + [pltpu.SemaphoreType.DMA] * (2 * nbuf),
    )
    def run(ids_hbm, table_hbm, out_hbm, idx_v, *bufs_and_sems):
        bufs = bufs_and_sems[:nbuf]
        sins = bufs_and_sems[nbuf:2 * nbuf]
        souts = bufs_and_sems[2 * nbuf:3 * nbuf]
        wid = lax.axis_index("s") * _NUM_CORES + lax.axis_index("c")
        base = wid * bpw
        pltpu.sync_copy(ids_hbm.at[pl.ds(base, bpw)], idx_v)

        def gather(g):
            k = g % nbuf
            return pltpu.make_async_copy(
                table_hbm.at[idx_v.at[pl.ds(g * chunk, chunk)]],
                bufs[k], sins[k])

        def put(g):
            k = g % nbuf
            return pltpu.make_async_copy(
                bufs[k], out_hbm.at[pl.ds(base + g * chunk, chunk)],
                souts[k])

        gathers, puts = {}, {}
        for g in range(min(2, nchunk)):
            gathers[g] = gather(g)
            gathers[g].start()
        for g in range(nchunk):
            if g + 1 < nchunk and g + 1 >= 2:
                if g - 2 >= 0:
                    puts[g - 2].wait()
                gathers[g + 1] = gather(g + 1)
                gathers[g + 1].start()
            gathers[g].wait()
            puts[g] = put(g)
            puts[g].start()
        for g in range(max(0, nchunk - 3), nchunk):
            puts[g].wait()

    return run(ids_flat, table)


def _tc_gather(ids_flat, table, *, rows_per_step=8):
    n = ids_flat.shape[0]
    d = table.shape[1]
    r = rows_per_step
    grid = (n // r,)

    def body(idx_ref, *refs):
        ins = refs[:r]
        out_ref = refs[r]
        for j in range(r):
            out_ref[j, :] = ins[j][0, 0, :]

    table3 = table.reshape(table.shape[0], 1, d)
    spec = pltpu.PrefetchScalarGridSpec(
        num_scalar_prefetch=1,
        grid=grid,
        in_specs=[
            pl.BlockSpec(
                (1, 1, d),
                functools.partial(
                    lambda j, i, idx_ref: (idx_ref[i * r + j], 0, 0), j))
            for j in range(r)
        ],
        out_specs=pl.BlockSpec((r, d), lambda i, idx_ref: (i, 0)),
    )
    return pl.pallas_call(
        body,
        grid_spec=spec,
        out_shape=jax.ShapeDtypeStruct((n, d), table.dtype),
    )(ids_flat, *([table3] * r))


_SC_ROWS = 0


def kernel(input_ids, embed_table):
    n = input_ids.size
    d = embed_table.shape[1]
    ids_flat = input_ids.reshape(-1).astype(jnp.int32)
    n_sc = _SC_ROWS
    parts = []
    if n_sc > 0:
        parts.append(_sc_gather(ids_flat[:n_sc], embed_table, n=n_sc, d=d))
    if n_sc < n:
        parts.append(_tc_gather(ids_flat[n_sc:], embed_table))
    out = parts[0] if len(parts) == 1 else jnp.concatenate(parts, axis=0)
    return out.reshape(input_ids.shape + (d,))


# speedup vs baseline: 17.7042x; 17.7042x over previous
import functools

import jax
import jax.numpy as jnp
from jax import lax
from jax.experimental import pallas as pl
from jax.experimental.pallas import tpu as pltpu
from jax.experimental.pallas import tpu_sc as plsc

_NUM_CORES = 2
_NUM_SUBCORES = 16
_NUM_WORKERS = _NUM_CORES * _NUM_SUBCORES


@functools.partial(jax.jit, static_argnames=("b", "s", "d"))
def _sc_gather(ids, table, *, b, s, d):
    n = b * s
    bpw = n // _NUM_WORKERS
    wpr = s // bpw
    chunk = 16
    nchunk = bpw // chunk
    mesh = plsc.VectorSubcoreMesh(core_axis_name="c", subcore_axis_name="s")

    @functools.partial(
        pl.kernel,
        mesh=mesh,
        out_type=jax.ShapeDtypeStruct((b, s, d), table.dtype),
        scratch_types=[
            pltpu.VMEM((bpw,), jnp.int32),
            pltpu.VMEM((chunk, d), table.dtype),
            pltpu.VMEM((chunk, d), table.dtype),
            pltpu.SemaphoreType.DMA,
            pltpu.SemaphoreType.DMA,
            pltpu.SemaphoreType.DMA,
            pltpu.SemaphoreType.DMA,
        ],
    )
    def run(ids_hbm, table_hbm, out_hbm, idx_v, buf0, buf1,
            sin0, sin1, sout0, sout1):
        wid = lax.axis_index("s") * _NUM_CORES + lax.axis_index("c")
        row = wid // wpr
        col = (wid % wpr) * bpw
        pltpu.sync_copy(ids_hbm.at[row, pl.ds(col, bpw)], idx_v)

        def gather(g, buf, sem):
            return pltpu.make_async_copy(
                table_hbm.at[idx_v.at[pl.ds(g * chunk, chunk)]], buf, sem)

        def put(g, buf, sem):
            return pltpu.make_async_copy(
                buf, out_hbm.at[row, pl.ds(col + g * chunk, chunk)], sem)

        gather(0, buf0, sin0).start()
        gather(1, buf1, sin1).start()
        gather(0, buf0, sin0).wait()
        put(0, buf0, sout0).start()

        def step(st, _):
            g1 = 2 * st + 1
            g2 = 2 * st + 2
            put(g2 - 2, buf0, sout0).wait()
            gather(g2, buf0, sin0).start()
            gather(g1, buf1, sin1).wait()
            put(g1, buf1, sout1).start()
            put(g1, buf1, sout1).wait()
            gather(g2 + 1, buf1, sin1).start()
            gather(g2, buf0, sin0).wait()
            put(g2, buf0, sout0).start()
            return ()

        lax.fori_loop(0, nchunk // 2 - 1, step, ())

        g_last = nchunk - 1
        gather(g_last, buf1, sin1).wait()
        put(g_last, buf1, sout1).start()
        put(g_last - 1, buf0, sout0).wait()
        put(g_last, buf1, sout1).wait()

    return run(ids, table)


def kernel(input_ids, embed_table):
    b, s = input_ids.shape
    d = embed_table.shape[1]
    ids = input_ids if input_ids.dtype == jnp.int32 else (
        input_ids.astype(jnp.int32))
    return _sc_gather(ids, embed_table, b=b, s=s, d=d)
